# baseline (speedup 1.0000x reference)
import functools

import jax
import jax.numpy as jnp
from jax import lax
from jax.experimental import pallas as pl
from jax.experimental.pallas import tpu as pltpu
from jax.experimental.pallas import tpu_sc as plsc

_NB = 4
_NV = 10000
_DY = 128
_NR, _ND = 4, 16
_NW = 32
_SPB = _NW // _NB
_RPW = _NV // _SPB
_NCHUNK = 10
_C = _RPW // _NCHUNK
_NSLOT = 3


@functools.lru_cache(maxsize=1)
def _build():
    mesh = plsc.VectorSubcoreMesh(core_axis_name="c", subcore_axis_name="s")

    @functools.partial(
        pl.kernel,
        out_type=(
            jax.ShapeDtypeStruct((_NB, _NV, _DY), jnp.float32),
            jax.ShapeDtypeStruct((_NB, _NV, _NR, _ND), jnp.int32),
            jax.ShapeDtypeStruct((_NB, _NV, _NR, _ND), jnp.float32),
        ),
        mesh=mesh,
        compiler_params=pltpu.CompilerParams(
            use_tc_tiling_on_sc=False, needs_layout_passes=False
        ),
        scratch_types=[
            pltpu.VMEM((_NCHUNK, _C), jnp.int32),
            pltpu.VMEM((_NV,), jnp.int32),
            pltpu.VMEM((_NSLOT, _C, _DY), jnp.float32),
            pltpu.VMEM((_NSLOT, _C, _NR, _ND), jnp.int32),
            pltpu.VMEM((_NSLOT, _C, _NR, _ND), jnp.float32),
            [pltpu.SemaphoreType.DMA] * _NSLOT,
            [pltpu.SemaphoreType.DMA] * _NSLOT,
        ],
    )
    def _shuffle(y_hbm, e_hbm, f_hbm, s_hbm, s3_hbm,
                 y_out, e_out, f_out,
                 idx_v, s_v, ybuf, ebuf, fbuf, gsem, ssem):
        wid = lax.axis_index("s") * 2 + lax.axis_index("c")
        b = wid // _SPB
        v0 = (wid % _SPB) * _RPW
        pltpu.sync_copy(s3_hbm.at[wid % _SPB], idx_v)
        pltpu.sync_copy(s_hbm, s_v)

        def _gather(c):
            k = c % _NSLOT
            ix = idx_v.at[c]
            return (
                pltpu.async_copy(y_hbm.at[b].at[ix], ybuf.at[k], gsem[k]),
                pltpu.async_copy(e_hbm.at[b].at[ix], ebuf.at[k], gsem[k]),
                pltpu.async_copy(f_hbm.at[b].at[ix], fbuf.at[k], gsem[k]),
            )

        def _scatter(c):
            k = c % _NSLOT
            dst = v0 + c * _C
            return (
                pltpu.async_copy(ybuf.at[k], y_out.at[b].at[pl.ds(dst, _C)], ssem[k]),
                pltpu.async_copy(ebuf.at[k], e_out.at[b].at[pl.ds(dst, _C)], ssem[k]),
                pltpu.async_copy(fbuf.at[k], f_out.at[b].at[pl.ds(dst, _C)], ssem[k]),
            )

        g = [None] * _NCHUNK
        sc = [None] * _NCHUNK
        g[0] = _gather(0)
        g[1] = _gather(1)
        for c in range(_NCHUNK):
            k = c % _NSLOT
            for d in g[c]:
                d.wait()

            def _remap(i, _):
                for r in range(_NR):
                    vals = ebuf[k, i, r, :]
                    ebuf[k, i, r, :] = plsc.load_gather(s_v, [vals])
                return 0

            lax.fori_loop(0, _C, _remap, 0)
            sc[c] = _scatter(c)
            if c + 2 < _NCHUNK:
                if c >= 1:
                    for d in sc[c - 1]:
                        d.wait()
                g[c + 2] = _gather(c + 2)
        for c in (_NCHUNK - 3, _NCHUNK - 2, _NCHUNK - 1):
            for d in sc[c]:
                d.wait()

    return _shuffle


def kernel(y, e, f):
    s = jax.random.permutation(jax.random.key(42), jnp.arange(_NV, dtype=jnp.int32))
    s3 = s.reshape(_SPB, _NCHUNK, _C)
    y2, e2, f2 = _build()(y, e, f, s, s3)
    return (y2, e2, f2, s, s)


# speedup vs baseline: 1.7466x; 1.7466x over previous
import functools

import jax
import jax.numpy as jnp
from jax import lax
from jax.experimental import pallas as pl
from jax.experimental.pallas import tpu as pltpu
from jax.experimental.pallas import tpu_sc as plsc

_NB = 4
_NV = 10000
_DY = 128
_NR, _ND = 4, 16
_NW = 32
_ROWS = _NB * _NV
_RPW = _ROWS // _NW
_NCHUNK = 10
_C = _RPW // _NCHUNK
_NSLOT = 3

_EFROWS = _NB * _NR * _ND
_STRIP = _EFROWS // _NW
_VC = 1024
_NVC = (_NV + _VC - 1) // _VC


@functools.lru_cache(maxsize=1)
def _build_y():
    mesh = plsc.VectorSubcoreMesh(core_axis_name="c", subcore_axis_name="s")

    @functools.partial(
        pl.kernel,
        out_type=jax.ShapeDtypeStruct((_ROWS, _DY), jnp.float32),
        mesh=mesh,
        compiler_params=pltpu.CompilerParams(
            use_tc_tiling_on_sc=False, needs_layout_passes=False
        ),
        scratch_types=[
            pltpu.VMEM((_NCHUNK, _C), jnp.int32),
            pltpu.VMEM((_NSLOT, _C, _DY), jnp.float32),
            [pltpu.SemaphoreType.DMA] * _NSLOT,
            [pltpu.SemaphoreType.DMA] * _NSLOT,
        ],
    )
    def _shuffle_y(y_hbm, idx_hbm, y_out, idx_v, ybuf, gsem, ssem):
        wid = lax.axis_index("s") * 2 + lax.axis_index("c")
        row0 = wid * _RPW
        pltpu.sync_copy(idx_hbm.at[wid], idx_v)

        def _gather(c):
            k = c % _NSLOT
            return pltpu.async_copy(y_hbm.at[idx_v.at[c]], ybuf.at[k], gsem[k])

        def _scatter(c):
            k = c % _NSLOT
            dst = row0 + c * _C
            return pltpu.async_copy(ybuf.at[k], y_out.at[pl.ds(dst, _C)], ssem[k])

        g = [None] * _NCHUNK
        sc = [None] * _NCHUNK
        g[0] = _gather(0)
        g[1] = _gather(1)
        for c in range(_NCHUNK):
            g[c].wait()
            sc[c] = _scatter(c)
            if c + 2 < _NCHUNK:
                if c >= 1:
                    sc[c - 1].wait()
                g[c + 2] = _gather(c + 2)
        for c in (_NCHUNK - 3, _NCHUNK - 2, _NCHUNK - 1):
            sc[c].wait()

    return _shuffle_y


@functools.lru_cache(maxsize=1)
def _build_ef():
    mesh = plsc.VectorSubcoreMesh(core_axis_name="c", subcore_axis_name="s")

    @functools.partial(
        pl.kernel,
        out_type=(
            jax.ShapeDtypeStruct((_EFROWS, _NV), jnp.int32),
            jax.ShapeDtypeStruct((_EFROWS, _NV), jnp.int32),
        ),
        mesh=mesh,
        compiler_params=pltpu.CompilerParams(
            use_tc_tiling_on_sc=False, needs_layout_passes=False
        ),
        scratch_types=[
            pltpu.VMEM((_NV,), jnp.int32),
            pltpu.VMEM((_STRIP, _NV), jnp.int32),
            pltpu.VMEM((2, _STRIP, _VC), jnp.int32),
            pltpu.SemaphoreType.DMA,
            [pltpu.SemaphoreType.DMA] * 2,
        ],
    )
    def _shuffle_ef(e_hbm, f_hbm, s_hbm, e_out, f_out, s_v, strip, obuf, gsem, ssem):
        wid = lax.axis_index("s") * 2 + lax.axis_index("c")
        r0 = wid * _STRIP
        pltpu.sync_copy(s_hbm, s_v)
        sc_prev = [None, None]

        def _do_array(src, dst, remap):
            pltpu.async_copy(src.at[pl.ds(r0, _STRIP)], strip, gsem).wait()
            for vc in range(_NVC):
                vbase = vc * _VC
                n = min(_VC, _NV - vbase)
                k = vc % 2
                if sc_prev[k] is not None:
                    sc_prev[k].wait()

                def _vec(j, _):
                    ixv = s_v[pl.ds(vbase + j * 16, 16)]
                    for r in range(_STRIP):
                        g = plsc.load_gather(strip.at[r], [ixv])
                        if remap:
                            g = plsc.load_gather(s_v, [g])
                        obuf[k, r, pl.ds(j * 16, 16)] = g
                    return 0

                lax.fori_loop(0, n // 16, _vec, 0)
                sc_prev[k] = pltpu.async_copy(
                    obuf.at[k, slice(None), pl.ds(0, n)],
                    dst.at[pl.ds(r0, _STRIP), pl.ds(vbase, n)],
                    ssem[k],
                )

        _do_array(e_hbm, e_out, True)
        _do_array(f_hbm, f_out, False)
        for d in sc_prev:
            if d is not None:
                d.wait()

    return _shuffle_ef


def _stage_s():
    return jax.random.permutation(
        jax.random.key(42), jnp.arange(_NV, dtype=jnp.int32)
    )


def kernel(y, e, f):
    s = _stage_s()
    idx = (jnp.arange(_NB, dtype=jnp.int32)[:, None] * _NV + s[None, :]).reshape(
        _NW, _NCHUNK, _C
    )
    y2 = _build_y()(y.reshape(_ROWS, _DY), idx).reshape(_NB, _NV, _DY)

    e2d = jnp.transpose(e, (0, 2, 3, 1)).reshape(_EFROWS, _NV)
    f2d = jax.lax.bitcast_convert_type(
        jnp.transpose(f, (0, 2, 3, 1)).reshape(_EFROWS, _NV), jnp.int32
    )
    e2o, f2o = _build_ef()(e2d, f2d, s)
    e2 = jnp.transpose(e2o.reshape(_NB, _NR, _ND, _NV), (0, 3, 1, 2))
    f2 = jnp.transpose(
        jax.lax.bitcast_convert_type(f2o, jnp.float32).reshape(_NB, _NR, _ND, _NV),
        (0, 3, 1, 2),
    )
    return (y2, e2, f2, s, s)
